# baseline (speedup 1.0000x reference)
import jax
import jax.numpy as jnp
from jax.experimental import pallas as pl


_M = 16
_N = 16
_P = 512
_Q = 512
_D = 64


def _kd_attn_kernel(q_ref, k_ref, v_ref, o_ref):
    q = q_ref[0, 0]
    k = k_ref[0, 0]
    v = v_ref[0, 0]
    q_rep = q.reshape(_M, _P, _D).sum(axis=0) * (1.0 / _M)
    k_rep = k.reshape(_N, _Q, _D).sum(axis=0) * (1.0 / _N)
    v_rep = v.reshape(_N, _Q, _D).sum(axis=0) * (1.0 / _N)
    scale = _D ** -0.5
    w = jax.lax.dot_general(
        q_rep, k_rep, (((1,), (1,)), ((), ())),
        preferred_element_type=jnp.float32) * scale
    w_max = jnp.max(w, axis=-1, keepdims=True)
    e = jnp.exp(w - w_max)
    soft = e / jnp.sum(e, axis=-1, keepdims=True)
    out_rep = jax.lax.dot_general(
        soft, v_rep, (((1,), (0,)), ((), ())),
        preferred_element_type=jnp.float32)
    o_ref[0, 0] = jnp.broadcast_to(out_rep[None], (_M, _P, _D)).reshape(_M * _P, _D)


def kernel(query, key, value, n_query_groups, n_key_groups):
    del n_query_groups, n_key_groups
    B, H, S, d = query.shape
    spec = pl.BlockSpec((1, 1, S, d), lambda b, h: (b, h, 0, 0))
    return pl.pallas_call(
        _kd_attn_kernel,
        grid=(B, H),
        in_specs=[spec, spec, spec],
        out_specs=spec,
        out_shape=jax.ShapeDtypeStruct((B, H, S, d), jnp.float32),
    )(query, key, value)


# speedup vs baseline: 1.3322x; 1.3322x over previous
import jax
import jax.numpy as jnp
from jax.experimental import pallas as pl


_M = 16
_N = 16
_P = 512
_Q = 512
_D = 64


def _kd_attn_kernel(q_ref, k_ref, v_ref, o_ref):
    q = q_ref[0]
    k = k_ref[0]
    v = v_ref[0]
    q_rep = q.reshape(_M, _P, _D).sum(axis=0) * (1.0 / _M)
    k_rep = k.reshape(_N, _Q, _D).sum(axis=0) * (1.0 / _N)
    v_rep = v.reshape(_N, _Q, _D).sum(axis=0) * (1.0 / _N)
    scale = _D ** -0.5
    w = jax.lax.dot_general(
        q_rep, k_rep, (((1,), (1,)), ((), ())),
        preferred_element_type=jnp.float32) * scale
    w_max = jnp.max(w, axis=-1, keepdims=True)
    e = jnp.exp(w - w_max)
    soft = e / jnp.sum(e, axis=-1, keepdims=True)
    out_rep = jax.lax.dot_general(
        soft, v_rep, (((1,), (0,)), ((), ())),
        preferred_element_type=jnp.float32)
    o_ref[0] = out_rep


def kernel(query, key, value, n_query_groups, n_key_groups):
    del n_query_groups, n_key_groups
    B, H, S, d = query.shape
    BH = B * H
    q = query.reshape(BH, S, d)
    k = key.reshape(BH, S, d)
    v = value.reshape(BH, S, d)
    in_spec = pl.BlockSpec((1, S, d), lambda i: (i, 0, 0))
    rep = pl.pallas_call(
        _kd_attn_kernel,
        grid=(BH,),
        in_specs=[in_spec, in_spec, in_spec],
        out_specs=pl.BlockSpec((1, _P, _D), lambda i: (i, 0, 0)),
        out_shape=jax.ShapeDtypeStruct((BH, _P, _D), jnp.float32),
    )(q, k, v)
    out = jnp.broadcast_to(rep[:, None], (BH, _M, _P, _D))
    return out.reshape(B, H, S, d)


# speedup vs baseline: 3.1464x; 2.3617x over previous
import jax
import jax.numpy as jnp
from jax.experimental import pallas as pl


_M = 16
_N = 16
_P = 512
_Q = 512
_D = 64


def _kd_attn_kernel(q_ref, k_ref, v_ref, o_ref):
    qT = q_ref[0]
    kT = k_ref[0]
    vT = v_ref[0]

    def group_mean(xT, n):
        acc = xT[:, 0:_Q]
        for g in range(1, n):
            acc = acc + xT[:, g * _Q:(g + 1) * _Q]
        return acc * (1.0 / n)

    q_repT = group_mean(qT, _M)
    k_repT = group_mean(kT, _N)
    v_repT = group_mean(vT, _N)
    scale = _D ** -0.5
    wT = jax.lax.dot_general(
        k_repT, q_repT, (((0,), (0,)), ((), ())),
        preferred_element_type=jnp.float32) * scale
    w_max = jnp.max(wT, axis=0, keepdims=True)
    e = jnp.exp(wT - w_max)
    softT = e / jnp.sum(e, axis=0, keepdims=True)
    o_ref[0] = jax.lax.dot_general(
        v_repT, softT, (((1,), (0,)), ((), ())),
        preferred_element_type=jnp.float32)


def kernel(query, key, value, n_query_groups, n_key_groups):
    del n_query_groups, n_key_groups
    B, H, S, d = query.shape
    BH = B * H
    qT = jnp.swapaxes(query, 2, 3).reshape(BH, d, S)
    kT = jnp.swapaxes(key, 2, 3).reshape(BH, d, S)
    vT = jnp.swapaxes(value, 2, 3).reshape(BH, d, S)
    in_spec = pl.BlockSpec((1, d, S), lambda i: (i, 0, 0))
    repT = pl.pallas_call(
        _kd_attn_kernel,
        grid=(BH,),
        in_specs=[in_spec, in_spec, in_spec],
        out_specs=pl.BlockSpec((1, d, _P), lambda i: (i, 0, 0)),
        out_shape=jax.ShapeDtypeStruct((BH, d, _P), jnp.float32),
    )(qT, kT, vT)
    outT = jnp.broadcast_to(repT[:, :, None, :], (BH, d, _M, _P))
    outT = outT.reshape(B, H, d, S)
    return jnp.swapaxes(outT, 2, 3)


# speedup vs baseline: 6.2021x; 1.9712x over previous
import jax
import jax.numpy as jnp
from jax.experimental import pallas as pl


_M = 16
_N = 16
_P = 512
_Q = 512
_D = 64


def _kd_attn_kernel(q_ref, k_ref, v_ref, o_ref):
    qT = q_ref[0]
    kT = k_ref[0]
    vT = v_ref[0]

    def group_mean(xT, n):
        acc = xT[:, 0:_Q]
        for g in range(1, n):
            acc = acc + xT[:, g * _Q:(g + 1) * _Q]
        return acc * (1.0 / n)

    q_repT = group_mean(qT, _M)
    k_repT = group_mean(kT, _N)
    v_repT = group_mean(vT, _N)
    scale = _D ** -0.5
    wT = jax.lax.dot_general(
        k_repT, q_repT, (((0,), (0,)), ((), ())),
        preferred_element_type=jnp.float32) * scale
    w_max = jnp.max(wT, axis=0, keepdims=True)
    e = jnp.exp(wT - w_max)
    softT = e / jnp.sum(e, axis=0, keepdims=True)
    out_repT = jax.lax.dot_general(
        v_repT, softT, (((1,), (0,)), ((), ())),
        preferred_element_type=jnp.float32)
    for g in range(_M):
        o_ref[0, :, g * _P:(g + 1) * _P] = out_repT


def kernel(query, key, value, n_query_groups, n_key_groups):
    del n_query_groups, n_key_groups
    B, H, S, d = query.shape
    BH = B * H
    qT = jnp.swapaxes(query, 2, 3).reshape(BH, d, S)
    kT = jnp.swapaxes(key, 2, 3).reshape(BH, d, S)
    vT = jnp.swapaxes(value, 2, 3).reshape(BH, d, S)
    in_spec = pl.BlockSpec((1, d, S), lambda i: (i, 0, 0))
    outT = pl.pallas_call(
        _kd_attn_kernel,
        grid=(BH,),
        in_specs=[in_spec, in_spec, in_spec],
        out_specs=pl.BlockSpec((1, d, S), lambda i: (i, 0, 0)),
        out_shape=jax.ShapeDtypeStruct((BH, d, S), jnp.float32),
    )(qT, kT, vT)
    return jnp.swapaxes(outT.reshape(B, H, d, S), 2, 3)


# speedup vs baseline: 6.2373x; 1.0057x over previous
import jax
import jax.numpy as jnp
from jax.experimental import pallas as pl


_M = 16
_N = 16


def _kd_attn_kernel(q_ref, k_ref, v_ref, o_ref):
    qT = q_ref[0]
    kT = k_ref[0]
    vT = v_ref[0]
    d, S = qT.shape
    p = S // _M

    def group_mean(xT, n):
        acc = xT[:, 0:p]
        for g in range(1, n):
            acc = acc + xT[:, g * p:(g + 1) * p]
        return acc * (1.0 / n)

    q_repT = group_mean(qT, _M)
    k_repT = group_mean(kT, _N)
    v_repT = group_mean(vT, _N)
    scale = d ** -0.5
    wT = jax.lax.dot_general(
        k_repT, q_repT, (((0,), (0,)), ((), ())),
        preferred_element_type=jnp.float32) * scale
    w_max = jnp.max(wT, axis=0, keepdims=True)
    e = jnp.exp(wT - w_max)
    softT = e / jnp.sum(e, axis=0, keepdims=True)
    out_repT = jax.lax.dot_general(
        v_repT, softT, (((1,), (0,)), ((), ())),
        preferred_element_type=jnp.float32)
    for g in range(_M):
        o_ref[0, :, g * p:(g + 1) * p] = out_repT


def kernel(query, key, value, n_query_groups, n_key_groups):
    del n_query_groups, n_key_groups
    B, H, S, d = query.shape
    BH = B * H
    qT = jnp.swapaxes(query, 2, 3).reshape(BH, d, S)
    kT = jnp.swapaxes(key, 2, 3).reshape(BH, d, S)
    vT = jnp.swapaxes(value, 2, 3).reshape(BH, d, S)
    in_spec = pl.BlockSpec((1, d, S), lambda i: (i, 0, 0))
    outT = pl.pallas_call(
        _kd_attn_kernel,
        grid=(BH,),
        in_specs=[in_spec, in_spec, in_spec],
        out_specs=pl.BlockSpec((1, d, S), lambda i: (i, 0, 0)),
        out_shape=jax.ShapeDtypeStruct((BH, d, S), jnp.float32),
    )(qT, kT, vT)
    return jnp.swapaxes(outT.reshape(B, H, d, S), 2, 3)
